# Initial kernel scaffold; baseline (speedup 1.0000x reference)
import jax
import jax.numpy as jnp
from jax.experimental import pallas as pl


def kernel(inputs, neighbors0, neighbors1, embed, W0, b0, W1):
    raise NotImplementedError("write your pallas kernel here")


# speedup vs baseline: 2.9255x; 2.9255x over previous
import functools

import jax
import jax.numpy as jnp
from jax import lax
from jax.experimental import pallas as pl
from jax.experimental.pallas import tpu as pltpu
from jax.experimental.pallas import tpu_sc as plsc

D = 128
LANES = 16
NVEC = D // LANES
N_INNER = 256
N_OUTER = 16
ROWS = 280


def _sc_make(B):
    NC, NS = 2, 16
    NW = NC * NS
    per = B // NW
    mesh = plsc.VectorSubcoreMesh(core_axis_name="c", subcore_axis_name="s")

    @functools.partial(
        pl.kernel,
        mesh=mesh,
        out_type=jax.ShapeDtypeStruct((B, 3 * D), jnp.float32),
        scratch_types=[
            pltpu.VMEM((per, ROWS), jnp.int32),
            pltpu.VMEM((ROWS, D), jnp.float32),
            pltpu.VMEM((per, 3 * D), jnp.float32),
            pltpu.SemaphoreType.DMA,
        ],
    )
    def sc_kernel(embed_hbm, idx_hbm, out_hbm, idx_v, rows_v, out_v, sem):
        wid = lax.axis_index("s") * NC + lax.axis_index("c")
        base = wid * per
        pltpu.sync_copy(idx_hbm.at[pl.ds(base, per)], idx_v)

        def elem(e, _):
            c0 = pltpu.async_copy(
                embed_hbm.at[idx_v.at[e, pl.ds(0, 128)]],
                rows_v.at[pl.ds(0, 128)], sem)
            c1 = pltpu.async_copy(
                embed_hbm.at[idx_v.at[e, pl.ds(128, 128)]],
                rows_v.at[pl.ds(128, 128)], sem)
            c2 = pltpu.async_copy(
                embed_hbm.at[idx_v.at[e, pl.ds(256, 24)]],
                rows_v.at[pl.ds(256, 24)], sem)
            c0.wait()
            c1.wait()
            c2.wait()

            zeros = tuple(jnp.zeros((LANES,), jnp.float32) for _ in range(NVEC))

            def red(i, acc):
                return tuple(acc[j] + rows_v[i, pl.ds(j * LANES, LANES)]
                             for j in range(NVEC))

            acc1 = lax.fori_loop(0, N_INNER, red, zeros)
            acc0 = lax.fori_loop(N_INNER, N_INNER + N_OUTER, red, zeros)
            for j in range(NVEC):
                out_v[e, pl.ds(j * LANES, LANES)] = acc1[j] * (1.0 / N_INNER)
                out_v[e, pl.ds(D + j * LANES, LANES)] = acc0[j] * (1.0 / N_OUTER)
                out_v[e, pl.ds(2 * D + j * LANES, LANES)] = \
                    rows_v[N_INNER + N_OUTER, pl.ds(j * LANES, LANES)]
            return 0

        lax.fori_loop(0, per, elem, 0)
        pltpu.sync_copy(out_v, out_hbm.at[pl.ds(base, per)])

    return sc_kernel


def _tc_dense(sc_out, W1, W0, b0):
    B = sc_out.shape[0]

    def body(sc_ref, w1_ref, w0_ref, b0_ref, out_ref):
        m1 = sc_ref[:, 0:D]
        m0 = sc_ref[:, D:2 * D]
        hv = sc_ref[:, 2 * D:3 * D]
        mean_n = (jnp.dot(m0, w1_ref[0:D, :], preferred_element_type=jnp.float32)
                  + jnp.dot(m1, w1_ref[D:2 * D, :], preferred_element_type=jnp.float32))
        z = (jnp.dot(hv, w0_ref[0:D, :], preferred_element_type=jnp.float32)
             + jnp.dot(mean_n, w0_ref[D:2 * D, :], preferred_element_type=jnp.float32)
             + b0_ref[:])
        out_ref[:] = jax.nn.sigmoid(z)

    return pl.pallas_call(
        body,
        out_shape=jax.ShapeDtypeStruct((B, D), jnp.float32),
    )(sc_out, W1, W0, b0)


def kernel(inputs, neighbors0, neighbors1, embed, W0, b0, W1):
    B = inputs.shape[0]
    idx = jnp.concatenate([
        neighbors1.reshape(B, N_INNER).astype(jnp.int32),
        neighbors0.reshape(B, N_OUTER).astype(jnp.int32),
        inputs.reshape(B, 1).astype(jnp.int32),
        jnp.zeros((B, ROWS - N_INNER - N_OUTER - 1), jnp.int32),
    ], axis=1)
    sc_out = _sc_make(B)(embed, idx)
    return _tc_dense(sc_out, W1, W0, b0.reshape(1, D))
